# baseline (speedup 1.0000x reference)
import functools

import jax
import jax.numpy as jnp
from jax import lax
from jax.experimental import pallas as pl
from jax.experimental.pallas import tpu as pltpu
from jax.experimental.pallas import tpu_sc as plsc

_E = 50000
_HID = 256
_OUT_C = 128

_NC = 2
_NS = 16
_NW = _NC * _NS
_CHUNK = 128
_EPAD = 50176
_ROWS = 4 * _EPAD
_RPW = _ROWS // _NW
_NCHUNK = _RPW // _CHUNK


def _make_gather(ncols):
    mesh = plsc.VectorSubcoreMesh(core_axis_name="c", subcore_axis_name="s")

    @functools.partial(
        pl.kernel,
        mesh=mesh,
        out_type=jax.ShapeDtypeStruct((_ROWS, ncols), jnp.float32),
        scratch_types=[
            pltpu.VMEM((_RPW,), jnp.int32),
            pltpu.VMEM((_CHUNK, ncols), jnp.float32),
            pltpu.VMEM((_CHUNK, ncols), jnp.float32),
            pltpu.SemaphoreType.DMA,
            pltpu.SemaphoreType.DMA,
        ],
    )
    def gather_k(tbl, idx, out, idx_v, buf0, buf1, sem0, sem1):
        wid = lax.axis_index("s") * _NC + lax.axis_index("c")
        base = wid * _RPW
        pltpu.sync_copy(idx.at[pl.ds(base, _RPW)], idx_v)

        bufs = (buf0, buf1)
        sems = (sem0, sem1)

        def _start(ci, slot):
            pltpu.async_copy(
                tbl.at[idx_v.at[pl.ds(ci * _CHUNK, _CHUNK)]],
                bufs[slot], sems[slot])

        def _finish(ci, slot):
            pltpu.make_async_copy(
                tbl.at[idx_v.at[pl.ds(ci * _CHUNK, _CHUNK)]],
                bufs[slot], sems[slot]).wait()
            pltpu.sync_copy(bufs[slot], out.at[pl.ds(base + ci * _CHUNK, _CHUNK)])

        _start(0, 0)

        def body(p, carry):
            ci = p * 2

            @pl.when(ci + 1 < _NCHUNK)
            def _():
                _start(ci + 1, 1)

            _finish(ci, 0)

            @pl.when(ci + 2 < _NCHUNK)
            def _():
                _start(ci + 2, 0)

            @pl.when(ci + 1 < _NCHUNK)
            def _():
                _finish(ci + 1, 1)

            return carry

        lax.fori_loop(0, (_NCHUNK + 1) // 2, body, 0)

    return gather_k


def _pack16(x):
    u = lax.bitcast_convert_type(x, jnp.uint32) + jnp.uint32(0x8000)
    w = (u[:, 128:] & jnp.uint32(0xFFFF0000)) | (u[:, :128] >> 16)
    return lax.bitcast_convert_type(w, jnp.float32)


def _unpack16(w):
    u = lax.bitcast_convert_type(w, jnp.uint32)
    lo = lax.bitcast_convert_type(u << 16, jnp.float32)
    hi = lax.bitcast_convert_type(u & jnp.uint32(0xFFFF0000), jnp.float32)
    return jnp.concatenate([lo, hi], axis=1)


def _combine(h, g_ref, packed):
    if packed:
        a = _unpack16(g_ref[0])
        b = _unpack16(g_ref[1])
        c = _unpack16(g_ref[2])
        d = _unpack16(g_ref[3])
    else:
        a = g_ref[0]
        b = g_ref[1]
        c = g_ref[2]
        d = g_ref[3]
    bf = jnp.bfloat16
    return (h.astype(bf), jnp.abs(a - c).astype(bf), (a + c).astype(bf),
            jnp.abs(b - d).astype(bf), (b + d).astype(bf))


def _matmul5(parts, w_ref, b_ref):
    acc = b_ref[...].astype(jnp.float32)
    off = 0
    for p in parts:
        k = p.shape[1]
        acc = acc + jnp.dot(p, w_ref[off:off + k, :],
                            preferred_element_type=jnp.float32)
        off += k
    return acc


def _conv_body(h_ref, g_ref, w_ref, b_ref, hb_ref, *, residual, packed):
    h = _unpack16(h_ref[...]) if packed else h_ref[...]
    acc = _matmul5(_combine(h, g_ref, packed), w_ref, b_ref)
    acc = jnp.maximum(acc, 0.0)
    if residual:
        acc += h
    hb_ref[...] = _pack16(acc)


def _final_body(h_ref, g_ref, w_ref, b_ref, wo_ref, bo_ref, out_ref):
    h = _unpack16(h_ref[...])
    acc = _matmul5(_combine(h, g_ref, True), w_ref, b_ref)
    h3 = jnp.maximum(acc, 0.0) + h
    out_ref[...] = jnp.dot(h3.astype(jnp.bfloat16), wo_ref[...],
                           preferred_element_type=jnp.float32) + bo_ref[...]


def _conv_call(h, g, w, b, *, residual, packed, be=1024):
    epad, k = h.shape
    gk = g.shape[2]
    kw = 5 * _HID if packed else k + 4 * gk
    return pl.pallas_call(
        functools.partial(_conv_body, residual=residual, packed=packed),
        grid=(epad // be,),
        in_specs=[
            pl.BlockSpec((be, k), lambda i: (i, 0)),
            pl.BlockSpec((4, be, gk), lambda i: (0, i, 0)),
            pl.BlockSpec((kw, _HID), lambda i: (0, 0)),
            pl.BlockSpec((1, _HID), lambda i: (0, 0)),
        ],
        out_specs=pl.BlockSpec((be, _HID // 2), lambda i: (i, 0)),
        out_shape=jax.ShapeDtypeStruct((epad, _HID // 2), jnp.float32),
        compiler_params=pltpu.CompilerParams(dimension_semantics=("arbitrary",)),
    )(h, g, w, b)


def _final_call(h, g, w, b, wo, bo, *, be=1024):
    epad, k = h.shape
    return pl.pallas_call(
        _final_body,
        grid=(epad // be,),
        in_specs=[
            pl.BlockSpec((be, k), lambda i: (i, 0)),
            pl.BlockSpec((4, be, _HID // 2), lambda i: (0, i, 0)),
            pl.BlockSpec((5 * _HID, _HID), lambda i: (0, 0)),
            pl.BlockSpec((1, _HID), lambda i: (0, 0)),
            pl.BlockSpec((_HID, _OUT_C), lambda i: (0, 0)),
            pl.BlockSpec((1, _OUT_C), lambda i: (0, 0)),
        ],
        out_specs=pl.BlockSpec((be, _OUT_C), lambda i: (i, 0)),
        out_shape=jax.ShapeDtypeStruct((_E, _OUT_C), jnp.float32),
        compiler_params=pltpu.CompilerParams(dimension_semantics=("arbitrary",)),
    )(h, g, w, b, wo, bo)


def kernel(x, edge_index, W0, b0, W1, b1, W2, b2, W3, b3, Wout, bout):
    bf = jnp.bfloat16
    idx = jnp.pad(edge_index[1].astype(jnp.int32).reshape(_E, 4).T,
                  ((0, 0), (0, _EPAD - _E))).reshape(-1)

    xp = jnp.pad(x, ((0, _EPAD - _E), (0, 123)))
    xs = jnp.pad(x, ((0, _EPAD - _E), (0, 11)))
    w0parts = W0.reshape(5, 5, _HID)
    w0p = jnp.concatenate([
        jnp.pad(w0parts[0], ((0, 11), (0, 0))),
        jnp.pad(w0parts[1], ((0, 123), (0, 0))),
        jnp.pad(w0parts[2], ((0, 123), (0, 0))),
        jnp.pad(w0parts[3], ((0, 123), (0, 0))),
        jnp.pad(w0parts[4], ((0, 123), (0, 0))),
    ])

    gather128 = _make_gather(128)

    g = gather128(xp, idx).reshape(4, _EPAD, 128)
    hb = _conv_call(xs, g, w0p.astype(bf), b0[None, :],
                    residual=False, packed=False)
    for wk, bk in ((W1, b1), (W2, b2)):
        g = gather128(hb, idx).reshape(4, _EPAD, 128)
        hb = _conv_call(hb, g, wk.astype(bf), bk[None, :],
                        residual=True, packed=True)
    g = gather128(hb, idx).reshape(4, _EPAD, 128)
    return _final_call(hb, g, W3.astype(bf), b3[None, :],
                       Wout.astype(bf), bout[None, :])


# speedup vs baseline: 1.0110x; 1.0110x over previous
import functools

import jax
import jax.numpy as jnp
from jax import lax
from jax.experimental import pallas as pl
from jax.experimental.pallas import tpu as pltpu
from jax.experimental.pallas import tpu_sc as plsc

_E = 50000
_HID = 256
_OUT_C = 128

_NC = 2
_NS = 16
_NW = _NC * _NS
_CHUNK = 128
_EPAD = 50176
_ROWS = 4 * _EPAD
_RPW = _ROWS // _NW
_NCHUNK = _RPW // _CHUNK


def _make_gather(ncols):
    mesh = plsc.VectorSubcoreMesh(core_axis_name="c", subcore_axis_name="s")

    nbuf = 4

    @functools.partial(
        pl.kernel,
        mesh=mesh,
        out_type=jax.ShapeDtypeStruct((_ROWS, ncols), jnp.float32),
        scratch_types=[
            pltpu.VMEM((_RPW,), jnp.int32),
        ] + [pltpu.VMEM((_CHUNK, ncols), jnp.float32) for _ in range(nbuf)]
          + [pltpu.SemaphoreType.DMA for _ in range(2 * nbuf)],
    )
    def gather_k(tbl, idx, out, idx_v, *scratch):
        bufs = scratch[:nbuf]
        gsems = scratch[nbuf:2 * nbuf]
        wsems = scratch[2 * nbuf:]
        wid = lax.axis_index("s") * _NC + lax.axis_index("c")
        base = wid * _RPW
        pltpu.sync_copy(idx.at[pl.ds(base, _RPW)], idx_v)

        def _gather_dma(ci, slot):
            return pltpu.make_async_copy(
                tbl.at[idx_v.at[pl.ds(ci * _CHUNK, _CHUNK)]],
                bufs[slot], gsems[slot])

        def _write_dma(ci, slot):
            return pltpu.make_async_copy(
                bufs[slot], out.at[pl.ds(base + ci * _CHUNK, _CHUNK)],
                wsems[slot])

        for s in range(nbuf - 1):
            _gather_dma(s, s).start()

        def body(p, carry):
            for s in range(nbuf):
                ci = p * nbuf + s

                @pl.when(ci < _NCHUNK)
                def _():
                    _gather_dma(ci, s).wait()
                    _write_dma(ci, s).start()

                nslot = (s + nbuf - 1) % nbuf

                @pl.when(ci + nbuf - 1 < _NCHUNK)
                def _():
                    @pl.when(ci >= 1)
                    def _():
                        _write_dma(ci - 1, nslot).wait()

                    _gather_dma(ci + nbuf - 1, nslot).start()

            return carry

        lax.fori_loop(0, (_NCHUNK + nbuf - 1) // nbuf, body, 0)
        for c in range(_NCHUNK - nbuf, _NCHUNK):
            _write_dma(c, c % nbuf).wait()

    return gather_k


def _pack16(x):
    u = lax.bitcast_convert_type(x, jnp.uint32) + jnp.uint32(0x8000)
    w = (u[:, 128:] & jnp.uint32(0xFFFF0000)) | (u[:, :128] >> 16)
    return lax.bitcast_convert_type(w, jnp.float32)


def _unpack16(w):
    u = lax.bitcast_convert_type(w, jnp.uint32)
    lo = lax.bitcast_convert_type(u << 16, jnp.float32)
    hi = lax.bitcast_convert_type(u & jnp.uint32(0xFFFF0000), jnp.float32)
    return jnp.concatenate([lo, hi], axis=1)


def _combine(h, g_ref, packed):
    if packed:
        a = _unpack16(g_ref[0])
        b = _unpack16(g_ref[1])
        c = _unpack16(g_ref[2])
        d = _unpack16(g_ref[3])
    else:
        a = g_ref[0]
        b = g_ref[1]
        c = g_ref[2]
        d = g_ref[3]
    bf = jnp.bfloat16
    return (h.astype(bf), jnp.abs(a - c).astype(bf), (a + c).astype(bf),
            jnp.abs(b - d).astype(bf), (b + d).astype(bf))


def _matmul5(parts, w_ref, b_ref):
    acc = b_ref[...].astype(jnp.float32)
    off = 0
    for p in parts:
        k = p.shape[1]
        acc = acc + jnp.dot(p, w_ref[off:off + k, :],
                            preferred_element_type=jnp.float32)
        off += k
    return acc


def _conv_body(h_ref, g_ref, w_ref, b_ref, hb_ref, *, residual, packed):
    h = _unpack16(h_ref[...]) if packed else h_ref[...]
    acc = _matmul5(_combine(h, g_ref, packed), w_ref, b_ref)
    acc = jnp.maximum(acc, 0.0)
    if residual:
        acc += h
    hb_ref[...] = _pack16(acc)


def _final_body(h_ref, g_ref, w_ref, b_ref, wo_ref, bo_ref, out_ref):
    h = _unpack16(h_ref[...])
    acc = _matmul5(_combine(h, g_ref, True), w_ref, b_ref)
    h3 = jnp.maximum(acc, 0.0) + h
    out_ref[...] = jnp.dot(h3.astype(jnp.bfloat16), wo_ref[...],
                           preferred_element_type=jnp.float32) + bo_ref[...]


def _conv_call(h, g, w, b, *, residual, packed, be=1024):
    epad, k = h.shape
    gk = g.shape[2]
    kw = 5 * _HID if packed else k + 4 * gk
    return pl.pallas_call(
        functools.partial(_conv_body, residual=residual, packed=packed),
        grid=(epad // be,),
        in_specs=[
            pl.BlockSpec((be, k), lambda i: (i, 0)),
            pl.BlockSpec((4, be, gk), lambda i: (0, i, 0)),
            pl.BlockSpec((kw, _HID), lambda i: (0, 0)),
            pl.BlockSpec((1, _HID), lambda i: (0, 0)),
        ],
        out_specs=pl.BlockSpec((be, _HID // 2), lambda i: (i, 0)),
        out_shape=jax.ShapeDtypeStruct((epad, _HID // 2), jnp.float32),
        compiler_params=pltpu.CompilerParams(dimension_semantics=("arbitrary",)),
    )(h, g, w, b)


def _final_call(h, g, w, b, wo, bo, *, be=1024):
    epad, k = h.shape
    return pl.pallas_call(
        _final_body,
        grid=(epad // be,),
        in_specs=[
            pl.BlockSpec((be, k), lambda i: (i, 0)),
            pl.BlockSpec((4, be, _HID // 2), lambda i: (0, i, 0)),
            pl.BlockSpec((5 * _HID, _HID), lambda i: (0, 0)),
            pl.BlockSpec((1, _HID), lambda i: (0, 0)),
            pl.BlockSpec((_HID, _OUT_C), lambda i: (0, 0)),
            pl.BlockSpec((1, _OUT_C), lambda i: (0, 0)),
        ],
        out_specs=pl.BlockSpec((be, _OUT_C), lambda i: (i, 0)),
        out_shape=jax.ShapeDtypeStruct((_E, _OUT_C), jnp.float32),
        compiler_params=pltpu.CompilerParams(dimension_semantics=("arbitrary",)),
    )(h, g, w, b, wo, bo)


def kernel(x, edge_index, W0, b0, W1, b1, W2, b2, W3, b3, Wout, bout):
    bf = jnp.bfloat16
    idx = jnp.pad(edge_index[1].astype(jnp.int32).reshape(_E, 4).T,
                  ((0, 0), (0, _EPAD - _E))).reshape(-1)

    xp = jnp.pad(x, ((0, _EPAD - _E), (0, 123)))
    xs = jnp.pad(x, ((0, _EPAD - _E), (0, 11)))
    w0parts = W0.reshape(5, 5, _HID)
    w0p = jnp.concatenate([
        jnp.pad(w0parts[0], ((0, 11), (0, 0))),
        jnp.pad(w0parts[1], ((0, 123), (0, 0))),
        jnp.pad(w0parts[2], ((0, 123), (0, 0))),
        jnp.pad(w0parts[3], ((0, 123), (0, 0))),
        jnp.pad(w0parts[4], ((0, 123), (0, 0))),
    ])

    gather128 = _make_gather(128)

    g = gather128(xp, idx).reshape(4, _EPAD, 128)
    hb = _conv_call(xs, g, w0p.astype(bf), b0[None, :],
                    residual=False, packed=False)
    for wk, bk in ((W1, b1), (W2, b2)):
        g = gather128(hb, idx).reshape(4, _EPAD, 128)
        hb = _conv_call(hb, g, wk.astype(bf), bk[None, :],
                        residual=True, packed=True)
    g = gather128(hb, idx).reshape(4, _EPAD, 128)
    return _final_call(hb, g, W3.astype(bf), b3[None, :],
                       Wout.astype(bf), bout[None, :])


# speedup vs baseline: 1.0290x; 1.0178x over previous
import functools

import jax
import jax.numpy as jnp
from jax import lax
from jax.experimental import pallas as pl
from jax.experimental.pallas import tpu as pltpu
from jax.experimental.pallas import tpu_sc as plsc

_E = 50000
_HID = 256
_OUT_C = 128

_NC = 2
_NS = 16
_NW = _NC * _NS
_CHUNK = 128
_EPAD = 50176
_ROWS = 4 * _EPAD
_RPW = _ROWS // _NW
_NCHUNK = _RPW // _CHUNK


def _make_gather(ncols, tc_tiling=True):
    mesh = plsc.VectorSubcoreMesh(core_axis_name="c", subcore_axis_name="s")

    nbuf = 4

    @functools.partial(
        pl.kernel,
        mesh=mesh,
        compiler_params=pltpu.CompilerParams(use_tc_tiling_on_sc=tc_tiling),
        out_type=jax.ShapeDtypeStruct((_ROWS, ncols), jnp.float32),
        scratch_types=[
            pltpu.VMEM((_RPW,), jnp.int32),
        ] + [pltpu.VMEM((_CHUNK, ncols), jnp.float32) for _ in range(nbuf)]
          + [pltpu.SemaphoreType.DMA for _ in range(2 * nbuf)],
    )
    def gather_k(tbl, idx, out, idx_v, *scratch):
        bufs = scratch[:nbuf]
        gsems = scratch[nbuf:2 * nbuf]
        wsems = scratch[2 * nbuf:]
        wid = lax.axis_index("s") * _NC + lax.axis_index("c")
        base = wid * _RPW
        pltpu.sync_copy(idx.at[pl.ds(base, _RPW)], idx_v)

        def _gather_dma(ci, slot):
            return pltpu.make_async_copy(
                tbl.at[idx_v.at[pl.ds(ci * _CHUNK, _CHUNK)]],
                bufs[slot], gsems[slot])

        def _write_dma(ci, slot):
            return pltpu.make_async_copy(
                bufs[slot], out.at[pl.ds(base + ci * _CHUNK, _CHUNK)],
                wsems[slot])

        for s in range(nbuf - 1):
            _gather_dma(s, s).start()

        def body(p, carry):
            for s in range(nbuf):
                ci = p * nbuf + s

                @pl.when(ci < _NCHUNK)
                def _():
                    _gather_dma(ci, s).wait()
                    _write_dma(ci, s).start()

                nslot = (s + nbuf - 1) % nbuf

                @pl.when(ci + nbuf - 1 < _NCHUNK)
                def _():
                    @pl.when(ci >= 1)
                    def _():
                        _write_dma(ci - 1, nslot).wait()

                    _gather_dma(ci + nbuf - 1, nslot).start()

            return carry

        lax.fori_loop(0, (_NCHUNK + nbuf - 1) // nbuf, body, 0)
        for c in range(_NCHUNK - nbuf, _NCHUNK):
            _write_dma(c, c % nbuf).wait()

    return gather_k


def _pack16(x):
    u = lax.bitcast_convert_type(x, jnp.uint32) + jnp.uint32(0x8000)
    w = (u[:, 128:] & jnp.uint32(0xFFFF0000)) | (u[:, :128] >> 16)
    return lax.bitcast_convert_type(w, jnp.float32)


def _unpack16(w):
    u = lax.bitcast_convert_type(w, jnp.uint32)
    lo = lax.bitcast_convert_type(u << 16, jnp.float32)
    hi = lax.bitcast_convert_type(u & jnp.uint32(0xFFFF0000), jnp.float32)
    return jnp.concatenate([lo, hi], axis=1)


def _combine(h, g_ref, packed):
    if packed:
        a = _unpack16(g_ref[0])
        b = _unpack16(g_ref[1])
        c = _unpack16(g_ref[2])
        d = _unpack16(g_ref[3])
    else:
        a = g_ref[0]
        b = g_ref[1]
        c = g_ref[2]
        d = g_ref[3]
    bf = jnp.bfloat16
    return (h.astype(bf), jnp.abs(a - c).astype(bf), (a + c).astype(bf),
            jnp.abs(b - d).astype(bf), (b + d).astype(bf))


def _matmul5(parts, w_ref, b_ref):
    acc = b_ref[...].astype(jnp.float32)
    off = 0
    for p in parts:
        k = p.shape[1]
        acc = acc + jnp.dot(p, w_ref[off:off + k, :],
                            preferred_element_type=jnp.float32)
        off += k
    return acc


def _conv_body(h_ref, g_ref, w_ref, b_ref, hb_ref, *, residual, packed):
    h = _unpack16(h_ref[...]) if packed else h_ref[...]
    acc = _matmul5(_combine(h, g_ref, packed), w_ref, b_ref)
    acc = jnp.maximum(acc, 0.0)
    if residual:
        acc += h
    hb_ref[...] = _pack16(acc)


def _final_body(h_ref, g_ref, w_ref, b_ref, wo_ref, bo_ref, out_ref):
    h = _unpack16(h_ref[...])
    acc = _matmul5(_combine(h, g_ref, True), w_ref, b_ref)
    h3 = jnp.maximum(acc, 0.0) + h
    out_ref[...] = jnp.dot(h3.astype(jnp.bfloat16), wo_ref[...],
                           preferred_element_type=jnp.float32) + bo_ref[...]


def _conv_call(h, g, w, b, *, residual, packed, be=1024):
    epad, k = h.shape
    gk = g.shape[2]
    kw = 5 * _HID if packed else k + 4 * gk
    return pl.pallas_call(
        functools.partial(_conv_body, residual=residual, packed=packed),
        grid=(epad // be,),
        in_specs=[
            pl.BlockSpec((be, k), lambda i: (i, 0)),
            pl.BlockSpec((4, be, gk), lambda i: (0, i, 0)),
            pl.BlockSpec((kw, _HID), lambda i: (0, 0)),
            pl.BlockSpec((1, _HID), lambda i: (0, 0)),
        ],
        out_specs=pl.BlockSpec((be, _HID // 2), lambda i: (i, 0)),
        out_shape=jax.ShapeDtypeStruct((epad, _HID // 2), jnp.float32),
        compiler_params=pltpu.CompilerParams(dimension_semantics=("arbitrary",)),
    )(h, g, w, b)


def _final_call(h, g, w, b, wo, bo, *, be=1024):
    epad, k = h.shape
    return pl.pallas_call(
        _final_body,
        grid=(epad // be,),
        in_specs=[
            pl.BlockSpec((be, k), lambda i: (i, 0)),
            pl.BlockSpec((4, be, _HID // 2), lambda i: (0, i, 0)),
            pl.BlockSpec((5 * _HID, _HID), lambda i: (0, 0)),
            pl.BlockSpec((1, _HID), lambda i: (0, 0)),
            pl.BlockSpec((_HID, _OUT_C), lambda i: (0, 0)),
            pl.BlockSpec((1, _OUT_C), lambda i: (0, 0)),
        ],
        out_specs=pl.BlockSpec((be, _OUT_C), lambda i: (i, 0)),
        out_shape=jax.ShapeDtypeStruct((_E, _OUT_C), jnp.float32),
        compiler_params=pltpu.CompilerParams(dimension_semantics=("arbitrary",)),
    )(h, g, w, b, wo, bo)


def kernel(x, edge_index, W0, b0, W1, b1, W2, b2, W3, b3, Wout, bout):
    bf = jnp.bfloat16
    idx = jnp.pad(edge_index[1].astype(jnp.int32).reshape(_E, 4).T,
                  ((0, 0), (0, _EPAD - _E))).reshape(-1)

    xs = jnp.pad(x, ((0, _EPAD - _E), (0, 11)))
    w0p = jnp.pad(W0.reshape(5, 5, _HID), ((0, 0), (0, 11), (0, 0))).reshape(80, _HID)

    gather16 = _make_gather(16, tc_tiling=False)
    gather128 = _make_gather(128)

    g = gather16(xs, idx).reshape(4, _EPAD, 16)
    hb = _conv_call(xs, g, w0p.astype(bf), b0[None, :],
                    residual=False, packed=False)
    for wk, bk in ((W1, b1), (W2, b2)):
        g = gather128(hb, idx).reshape(4, _EPAD, 128)
        hb = _conv_call(hb, g, wk.astype(bf), bk[None, :],
                        residual=True, packed=True)
    g = gather128(hb, idx).reshape(4, _EPAD, 128)
    return _final_call(hb, g, W3.astype(bf), b3[None, :],
                       Wout.astype(bf), bout[None, :])
